# baseline (speedup 1.0000x reference)
import numpy as np
import jax
import jax.numpy as jnp
from jax import lax
from jax.experimental import pallas as pl
from jax.experimental.pallas import tpu as pltpu
from jax.experimental.pallas import tpu_sc as plsc

_STRIDE = 16
_NEG_OVERLAP = 0.3
_POS_OVERLAP = 0.7
_RPN_BATCHSIZE = 256
_NUM_FG = 128
_FH = _FW = 64
_G = 100
_GP = 112
_A = _FH * _FW * 9
_NW = 16
_NPW = _A // _NW
_NV = _NPW // 16
_KV = _A // 16
_NMERGE = 13


def _np_base_anchors(base_size=16, ratios=(0.5, 1.0, 2.0), scales=(8, 16, 32)):
    base = np.array([1, 1, base_size, base_size], dtype=np.float32) - 1
    w = base[2] - base[0] + 1
    h = base[3] - base[1] + 1
    x_ctr = base[0] + 0.5 * (w - 1)
    y_ctr = base[1] + 0.5 * (h - 1)
    size = w * h
    anchors = []
    for r in ratios:
        size_r = size / r
        ws = np.round(np.sqrt(size_r))
        hs = np.round(ws * r)
        for s in scales:
            ws2 = ws * s
            hs2 = hs * s
            anchors.append([x_ctr - 0.5 * (ws2 - 1), y_ctr - 0.5 * (hs2 - 1),
                            x_ctr + 0.5 * (ws2 - 1), y_ctr + 0.5 * (hs2 - 1)])
    return np.array(anchors, dtype=np.float32)


def _np_all_anchors(fh, fw, stride, base):
    sx = np.arange(fw, dtype=np.float32) * stride
    sy = np.arange(fh, dtype=np.float32) * stride
    sx, sy = np.meshgrid(sx, sy)
    shifts = np.stack([sx.ravel(), sy.ravel(), sx.ravel(), sy.ravel()],
                      axis=1).astype(np.float32)
    all_a = base[None, :, :] + shifts[:, None, :]
    return all_a.reshape(-1, 4)


_ANCHORS = _np_all_anchors(_FH, _FW, _STRIDE, _np_base_anchors())


def _tf2x32(k1, k2, x0, x1):
    def rotl(x, d):
        return (x << np.uint32(d)) | (x >> np.uint32(32 - d))
    ks0 = np.uint32(k1)
    ks1 = np.uint32(k2)
    ks2 = ks0 ^ ks1 ^ np.uint32(0x1BD11BDA)
    x0 = (x0 + ks0).astype(np.uint32)
    x1 = (x1 + ks1).astype(np.uint32)
    rots = ((13, 15, 26, 6), (17, 29, 16, 24))
    inject = ((ks1, ks2, 1), (ks2, ks0, 2), (ks0, ks1, 3),
              (ks1, ks2, 4), (ks2, ks0, 5))
    for blk in range(5):
        for r in rots[blk % 2]:
            x0 = (x0 + x1).astype(np.uint32)
            x1 = rotl(x1, r)
            x1 = x1 ^ x0
        a, b, c = inject[blk]
        x0 = (x0 + a).astype(np.uint32)
        x1 = (x1 + b + np.uint32(c)).astype(np.uint32)
    return x0, x1


def _key42_uniforms(n):
    b1, b2 = _tf2x32(0, 42, np.zeros(2, np.uint32),
                     np.arange(2, dtype=np.uint32))
    out = []
    for (c1, c2) in ((b1[0], b2[0]), (b1[1], b2[1])):
        h1, h2 = _tf2x32(c1, c2, np.zeros(n, np.uint32),
                         np.arange(n, dtype=np.uint32))
        bits = h1 ^ h2
        f = ((bits >> np.uint32(9)) | np.uint32(0x3F800000)).view(np.float32)
        out.append(np.maximum(np.float32(0.0), f - np.float32(1.0)))
    return out


_RND_F, _RND_B = _key42_uniforms(_A)
_ORD_F = np.argsort(_RND_F, kind="stable").astype(np.int32)
_ORD_B = np.argsort(_RND_B, kind="stable").astype(np.int32)

_mgrid = np.linspace(1.0, 2.0, 8193)
_LOG2C = [float(c) for c in np.polyfit(_mgrid, np.log2(_mgrid), 7)]
_LN2 = float(np.log(2.0))


def _plog(x):
    b = lax.bitcast_convert_type(x, jnp.int32)
    e = (b >> 23) - 127
    m = lax.bitcast_convert_type((b & 0x007FFFFF) | 0x3F800000, jnp.float32)
    p = jnp.float32(_LOG2C[0])
    for c in _LOG2C[1:]:
        p = p * m + jnp.float32(c)
    return (e.astype(jnp.float32) + p) * jnp.float32(_LN2)


def _sc_body(ax1_h, ay1_h, ax2_h, ay2_h, ordf_h, ordb_h, gt_h, meta_h,
             lab_h, dx_h, dy_h, dw_h, dh_h,
             ax1, ay1, ax2, ay2, ins, bv, bj, cm, ca, gt, metav,
             mbv, mba, sidx, winb, hitb, labb, fgb, bgb, kfb, kbb,
             dxb, dyb, dwb, dhb, bigord, bigflag, sgt,
             sh_cm, sh_ca, sh_win, sh_ff, sh_fb):
    f32 = jnp.float32
    i32 = jnp.int32
    wid = lax.axis_index("s")
    base = wid * _NPW
    iota = lax.iota(i32, 16)

    pltpu.sync_copy(ax1_h.at[pl.ds(base, _NPW)], ax1)
    pltpu.sync_copy(ay1_h.at[pl.ds(base, _NPW)], ay1)
    pltpu.sync_copy(ax2_h.at[pl.ds(base, _NPW)], ax2)
    pltpu.sync_copy(ay2_h.at[pl.ds(base, _NPW)], ay2)
    pltpu.sync_copy(gt_h, gt)
    pltpu.sync_copy(meta_h, metav)

    mv = metav[...]
    m_h = mv[0]
    m_w = mv[1]

    for r in range(4):
        for k in range(_GP // 16):
            vec = gt[pl.ds(r * _GP + k * 16, 16)]
            for l in range(16):
                j = k * 16 + l
                if j < _G:
                    sgt[r * _G + j] = vec[l]

    def initj(k, _):
        cm[pl.ds(k * 16, 16)] = jnp.full((16,), -2.0, f32)
        ca[pl.ds(k * 16, 16)] = jnp.zeros((16,), i32)
        return 0

    lax.fori_loop(0, _GP, initj, 0)

    def zt(k, _):
        hitb[pl.ds(k * 16, 16)] = jnp.zeros((16,), f32)
        return 0

    lax.fori_loop(0, _NV, zt, 0)
    sidx[...] = jnp.full((16,), _A, i32)

    def vi_body(vi, _):
        o = vi * 16
        a1 = ax1[pl.ds(o, 16)]
        a2 = ay1[pl.ds(o, 16)]
        a3 = ax2[pl.ds(o, 16)]
        a4 = ay2[pl.ds(o, 16)]
        aw = a3 - a1 + 1.0
        ah = a4 - a2 + 1.0
        area = aw * ah
        insw = (jnp.where(a1 >= 0.0, 1, 0) * jnp.where(a2 >= 0.0, 1, 0)
                * jnp.where(a3 < m_w, 1, 0) * jnp.where(a4 < m_h, 1, 0))
        ins[pl.ds(o, 16)] = insw
        aidx = base + o + iota

        def j_body(jh, carry):
            bvv, bjv = carry
            for j in (jh * 4, jh * 4 + 1, jh * 4 + 2, jh * 4 + 3):
                gx1 = sgt[j]
                gy1 = sgt[_G + j]
                gx2 = sgt[2 * _G + j]
                gy2 = sgt[3 * _G + j]
                gw = gx2 - gx1 + 1.0
                gh = gy2 - gy1 + 1.0
                garea = gw * gh
                iw = jnp.minimum(a3, gx2) - jnp.maximum(a1, gx1) + 1.0
                ih = jnp.minimum(a4, gy2) - jnp.maximum(a2, gy1) + 1.0
                iw = jnp.maximum(iw, 0.0)
                ih = jnp.maximum(ih, 0.0)
                inter = iw * ih
                union = area + garea - inter
                iou = inter / union
                masked = jnp.where(insw > 0, iou, -1.0)
                c = masked > bvv
                bvv = jnp.where(c, masked, bvv)
                bjv = jnp.where(c, j, bjv)
                jo = j * 16
                cmo = cm[pl.ds(jo, 16)]
                cao = ca[pl.ds(jo, 16)]
                cc = masked > cmo
                cm[pl.ds(jo, 16)] = jnp.where(cc, masked, cmo)
                ca[pl.ds(jo, 16)] = jnp.where(cc, aidx, cao)
            return (bvv, bjv)

        bvv, bjv = lax.fori_loop(
            0, _G // 4, j_body,
            (jnp.full((16,), -2.0, f32), jnp.zeros((16,), i32)))
        bv[pl.ds(o, 16)] = bvv
        bj[pl.ds(o, 16)] = bjv
        return 0

    lax.fori_loop(0, _NV, vi_body, 0)

    pltpu.sync_copy(cm, sh_cm.at[pl.ds(wid * _GP * 16, _GP * 16)])
    pltpu.sync_copy(ca, sh_ca.at[pl.ds(wid * _GP * 16, _GP * 16)])
    plsc.subcore_barrier()

    @pl.when(wid < _NMERGE)
    def _merge():
        j0 = wid * 8

        def fetch(s, _):
            pltpu.sync_copy(sh_cm.at[pl.ds(s * _GP * 16 + wid * 128, 128)],
                            mbv.at[pl.ds(s * 128, 128)])
            pltpu.sync_copy(sh_ca.at[pl.ds(s * _GP * 16 + wid * 128, 128)],
                            mba.at[pl.ds(s * 128, 128)])
            return 0

        lax.fori_loop(0, _NW, fetch, 0)
        idxv = jnp.full((16,), _A, i32)
        for jj in range(8):
            def mbody(s, carry):
                gmax, garg = carry
                vv = mbv[pl.ds(s * 128 + jj * 16, 16)]
                va = mba[pl.ds(s * 128 + jj * 16, 16)]
                cc = vv > gmax
                return (jnp.where(cc, vv, gmax), jnp.where(cc, va, garg))

            gmax, garg = lax.fori_loop(
                0, _NW, mbody,
                (jnp.full((16,), -3.0, f32), jnp.zeros((16,), i32)))
            mx = plsc.cummax(gmax)[15]
            am = -plsc.cummax(jnp.where(gmax == mx, -garg, -_A))[15]
            sel = (jnp.where(iota == jj, 1, 0)
                   * jnp.where(j0 + jj < _G, 1, 0))
            idxv = jnp.where(sel > 0, am, idxv)
        sidx[...] = idxv

    pltpu.sync_copy(sidx, sh_win.at[pl.ds(wid * 16, 16)])
    plsc.subcore_barrier()

    pltpu.sync_copy(sh_win, winb)
    ones16 = jnp.ones((16,), f32)
    for r in range(_NMERGE):
        wv = winb[pl.ds(r * 16, 16)]
        loc = wv - base
        mask = (jnp.where(loc >= 0, 1, 0) * jnp.where(loc < _NPW, 1, 0)) > 0
        locc = jnp.clip(loc, 0, _NPW - 1)
        plsc.store_scatter(hitb, [locc], ones16, mask=mask)

    def lab_body(vi, _):
        o = vi * 16
        insw = ins[pl.ds(o, 16)]
        bvv = bv[pl.ds(o, 16)]
        hv = hitb[pl.ds(o, 16)]
        lv = jnp.where(insw * jnp.where(bvv < _NEG_OVERLAP, 1, 0) > 0,
                       0.0, -1.0)
        lv = jnp.where(hv > 0.5, 1.0, lv)
        lv = jnp.where(insw * jnp.where(bvv >= _POS_OVERLAP, 1, 0) > 0,
                       1.0, lv)
        lv = jnp.where(insw > 0, lv, -1.0)
        labb[pl.ds(o, 16)] = lv
        fgb[pl.ds(o, 16)] = jnp.where(lv == 1.0, 1, 0)
        bgb[pl.ds(o, 16)] = jnp.where(lv == 0.0, 1, 0)
        return 0

    lax.fori_loop(0, _NV, lab_body, 0)
    pltpu.sync_copy(fgb, sh_ff.at[pl.ds(base, _NPW)])
    pltpu.sync_copy(bgb, sh_fb.at[pl.ds(base, _NPW)])
    plsc.subcore_barrier()

    @pl.when(wid == 0)
    def _subsample():
        def one_pass(ord_h, sh_flag, target):
            pltpu.sync_copy(ord_h, bigord)
            pltpu.sync_copy(sh_flag, bigflag)

            def kbody(k4, carry):
                k = k4 * 8
                idxs = [bigord[pl.ds((k + u) * 16, 16)] for u in range(8)]
                fs = [plsc.load_gather(bigflag, [ix]) for ix in idxs]
                cums = [plsc.cumsum(f) for f in fs]
                for u in range(8):
                    rank = cums[u] + carry
                    kept = (jnp.where(fs[u] > 0, 1, 0)
                            * jnp.where(rank <= target, 1, 0))
                    plsc.store_scatter(bigflag, [idxs[u]], kept)
                    carry = carry + cums[u][15]
                return carry

            total = lax.fori_loop(0, _KV // 8, kbody, jnp.int32(0))
            pltpu.sync_copy(bigflag, sh_flag)
            return total

        nf = one_pass(ordf_h, sh_ff, jnp.int32(_NUM_FG))
        nbg = _RPN_BATCHSIZE - jnp.minimum(nf, _NUM_FG)
        one_pass(ordb_h, sh_fb, nbg)

    plsc.subcore_barrier()

    pltpu.sync_copy(sh_ff.at[pl.ds(base, _NPW)], kfb)
    pltpu.sync_copy(sh_fb.at[pl.ds(base, _NPW)], kbb)

    def fin_body(vi, _):
        o = vi * 16
        lv = labb[pl.ds(o, 16)]
        kf_ = kfb[pl.ds(o, 16)]
        kb_ = kbb[pl.ds(o, 16)]
        lv = jnp.where(jnp.where(lv == 1.0, 1, 0) * jnp.where(kf_ == 0, 1, 0)
                       > 0, -1.0, lv)
        lv = jnp.where(jnp.where(lv == 0.0, 1, 0) * jnp.where(kb_ == 0, 1, 0)
                       > 0, -1.0, lv)
        labb[pl.ds(o, 16)] = lv
        bjv = bj[pl.ds(o, 16)]
        gx1 = plsc.load_gather(gt, [bjv])
        gy1 = plsc.load_gather(gt, [bjv + _GP])
        gx2 = plsc.load_gather(gt, [bjv + 2 * _GP])
        gy2 = plsc.load_gather(gt, [bjv + 3 * _GP])
        gw = gx2 - gx1 + 1.0
        gh = gy2 - gy1 + 1.0
        gcx = gx1 + 0.5 * gw
        gcy = gy1 + 0.5 * gh
        a1 = ax1[pl.ds(o, 16)]
        a2 = ay1[pl.ds(o, 16)]
        a3 = ax2[pl.ds(o, 16)]
        a4 = ay2[pl.ds(o, 16)]
        aw = a3 - a1 + 1.0
        ah = a4 - a2 + 1.0
        acx = a1 + 0.5 * aw
        acy = a2 + 0.5 * ah
        insw = ins[pl.ds(o, 16)]
        dxb[pl.ds(o, 16)] = jnp.where(insw > 0, (gcx - acx) / aw, 0.0)
        dyb[pl.ds(o, 16)] = jnp.where(insw > 0, (gcy - acy) / ah, 0.0)
        dwb[pl.ds(o, 16)] = jnp.where(insw > 0, _plog(gw / aw), 0.0)
        dhb[pl.ds(o, 16)] = jnp.where(insw > 0, _plog(gh / ah), 0.0)
        return 0

    lax.fori_loop(0, _NV, fin_body, 0)

    pltpu.sync_copy(labb, lab_h.at[pl.ds(base, _NPW)])
    pltpu.sync_copy(dxb, dx_h.at[pl.ds(base, _NPW)])
    pltpu.sync_copy(dyb, dy_h.at[pl.ds(base, _NPW)])
    pltpu.sync_copy(dwb, dw_h.at[pl.ds(base, _NPW)])
    pltpu.sync_copy(dhb, dh_h.at[pl.ds(base, _NPW)])


def _make_sc_kernel():
    f32 = jnp.float32
    i32 = jnp.int32
    return pl.kernel(
        _sc_body,
        out_type=[jax.ShapeDtypeStruct((_A,), f32) for _ in range(5)],
        mesh=plsc.VectorSubcoreMesh(core_axis_name="c", subcore_axis_name="s",
                                    num_cores=1),
        compiler_params=pltpu.CompilerParams(needs_layout_passes=False),
        scratch_types=[
            pltpu.VMEM((_NPW,), f32),
            pltpu.VMEM((_NPW,), f32),
            pltpu.VMEM((_NPW,), f32),
            pltpu.VMEM((_NPW,), f32),
            pltpu.VMEM((_NPW,), i32),
            pltpu.VMEM((_NPW,), f32),
            pltpu.VMEM((_NPW,), i32),
            pltpu.VMEM((_GP * 16,), f32),
            pltpu.VMEM((_GP * 16,), i32),
            pltpu.VMEM((4 * _GP,), f32),
            pltpu.VMEM((16,), f32),
            pltpu.VMEM((_NW * 128,), f32),
            pltpu.VMEM((_NW * 128,), i32),
            pltpu.VMEM((16,), i32),
            pltpu.VMEM((_NW * 16,), i32),
            pltpu.VMEM((_NPW,), f32),
            pltpu.VMEM((_NPW,), f32),
            pltpu.VMEM((_NPW,), i32),
            pltpu.VMEM((_NPW,), i32),
            pltpu.VMEM((_NPW,), i32),
            pltpu.VMEM((_NPW,), i32),
            pltpu.VMEM((_NPW,), f32),
            pltpu.VMEM((_NPW,), f32),
            pltpu.VMEM((_NPW,), f32),
            pltpu.VMEM((_NPW,), f32),
            pltpu.VMEM((_A,), i32),
            pltpu.VMEM((_A,), i32),
            pltpu.SMEM((4 * _G,), f32),
            pltpu.VMEM_SHARED((_NW * _GP * 16,), f32),
            pltpu.VMEM_SHARED((_NW * _GP * 16,), i32),
            pltpu.VMEM_SHARED((_NW * 16,), i32),
            pltpu.VMEM_SHARED((_A,), i32),
            pltpu.VMEM_SHARED((_A,), i32),
        ],
    )


def kernel(scores, gt_boxes, metadata):
    del scores
    f32 = jnp.float32
    gtt = jnp.zeros((4, _GP), f32).at[:, :_G].set(
        gt_boxes.T.astype(f32)).reshape(-1)
    metap = jnp.zeros((16,), f32).at[:3].set(metadata.astype(f32))
    fn = _make_sc_kernel()
    lab, dx, dy, dw, dh = fn(
        jnp.asarray(_ANCHORS[:, 0]), jnp.asarray(_ANCHORS[:, 1]),
        jnp.asarray(_ANCHORS[:, 2]), jnp.asarray(_ANCHORS[:, 3]),
        jnp.asarray(_ORD_F), jnp.asarray(_ORD_B), gtt, metap)
    return jnp.stack([lab, dx, dy, dw, dh], axis=1)


# speedup vs baseline: 1.1512x; 1.1512x over previous
import numpy as np
import jax
import jax.numpy as jnp
from jax import lax
from jax.experimental import pallas as pl
from jax.experimental.pallas import tpu as pltpu
from jax.experimental.pallas import tpu_sc as plsc

_STRIDE = 16
_NEG_OVERLAP = 0.3
_POS_OVERLAP = 0.7
_RPN_BATCHSIZE = 256
_NUM_FG = 128
_FH = _FW = 64
_G = 100
_GP = 112
_A = _FH * _FW * 9
_NW = 16
_NPW = _A // _NW
_NV = _NPW // 16
_KV = _A // 16
_NMERGE = 13


def _np_base_anchors(base_size=16, ratios=(0.5, 1.0, 2.0), scales=(8, 16, 32)):
    base = np.array([1, 1, base_size, base_size], dtype=np.float32) - 1
    w = base[2] - base[0] + 1
    h = base[3] - base[1] + 1
    x_ctr = base[0] + 0.5 * (w - 1)
    y_ctr = base[1] + 0.5 * (h - 1)
    size = w * h
    anchors = []
    for r in ratios:
        size_r = size / r
        ws = np.round(np.sqrt(size_r))
        hs = np.round(ws * r)
        for s in scales:
            ws2 = ws * s
            hs2 = hs * s
            anchors.append([x_ctr - 0.5 * (ws2 - 1), y_ctr - 0.5 * (hs2 - 1),
                            x_ctr + 0.5 * (ws2 - 1), y_ctr + 0.5 * (hs2 - 1)])
    return np.array(anchors, dtype=np.float32)


def _np_all_anchors(fh, fw, stride, base):
    sx = np.arange(fw, dtype=np.float32) * stride
    sy = np.arange(fh, dtype=np.float32) * stride
    sx, sy = np.meshgrid(sx, sy)
    shifts = np.stack([sx.ravel(), sy.ravel(), sx.ravel(), sy.ravel()],
                      axis=1).astype(np.float32)
    all_a = base[None, :, :] + shifts[:, None, :]
    return all_a.reshape(-1, 4)


_ANCHORS = _np_all_anchors(_FH, _FW, _STRIDE, _np_base_anchors())


def _tf2x32(k1, k2, x0, x1):
    def rotl(x, d):
        return (x << np.uint32(d)) | (x >> np.uint32(32 - d))
    ks0 = np.uint32(k1)
    ks1 = np.uint32(k2)
    ks2 = ks0 ^ ks1 ^ np.uint32(0x1BD11BDA)
    x0 = (x0 + ks0).astype(np.uint32)
    x1 = (x1 + ks1).astype(np.uint32)
    rots = ((13, 15, 26, 6), (17, 29, 16, 24))
    inject = ((ks1, ks2, 1), (ks2, ks0, 2), (ks0, ks1, 3),
              (ks1, ks2, 4), (ks2, ks0, 5))
    for blk in range(5):
        for r in rots[blk % 2]:
            x0 = (x0 + x1).astype(np.uint32)
            x1 = rotl(x1, r)
            x1 = x1 ^ x0
        a, b, c = inject[blk]
        x0 = (x0 + a).astype(np.uint32)
        x1 = (x1 + b + np.uint32(c)).astype(np.uint32)
    return x0, x1


def _key42_uniforms(n):
    b1, b2 = _tf2x32(0, 42, np.zeros(2, np.uint32),
                     np.arange(2, dtype=np.uint32))
    out = []
    for (c1, c2) in ((b1[0], b2[0]), (b1[1], b2[1])):
        h1, h2 = _tf2x32(c1, c2, np.zeros(n, np.uint32),
                         np.arange(n, dtype=np.uint32))
        bits = h1 ^ h2
        f = ((bits >> np.uint32(9)) | np.uint32(0x3F800000)).view(np.float32)
        out.append(np.maximum(np.float32(0.0), f - np.float32(1.0)))
    return out


_RND_F, _RND_B = _key42_uniforms(_A)
_ORD_F = np.argsort(_RND_F, kind="stable").astype(np.int32)
_ORD_B = np.argsort(_RND_B, kind="stable").astype(np.int32)

_mgrid = np.linspace(1.0, 2.0, 8193)
_LOG2C = [float(c) for c in np.polyfit(_mgrid, np.log2(_mgrid), 7)]
_LN2 = float(np.log(2.0))


def _plog(x):
    b = lax.bitcast_convert_type(x, jnp.int32)
    e = (b >> 23) - 127
    m = lax.bitcast_convert_type((b & 0x007FFFFF) | 0x3F800000, jnp.float32)
    p = jnp.float32(_LOG2C[0])
    for c in _LOG2C[1:]:
        p = p * m + jnp.float32(c)
    return (e.astype(jnp.float32) + p) * jnp.float32(_LN2)


def _sc_body(ax1_h, ay1_h, ax2_h, ay2_h, ordf_h, ordb_h, gt_h, meta_h,
             lab_h, dx_h, dy_h, dw_h, dh_h,
             ax1, ay1, ax2, ay2, ins, bv, bj, cm, ca, gt, metav,
             mbv, mba, sidx, winb, hitb, labb, fgb, bgb, kfb, kbb,
             dxb, dyb, dwb, dhb, bigord, bigflag, sgt,
             sh_cm, sh_ca, sh_win, sh_ff, sh_fb):
    f32 = jnp.float32
    i32 = jnp.int32
    wid = lax.axis_index("s")
    base = wid * _NPW
    iota = lax.iota(i32, 16)

    pltpu.sync_copy(ax1_h.at[pl.ds(base, _NPW)], ax1)
    pltpu.sync_copy(ay1_h.at[pl.ds(base, _NPW)], ay1)
    pltpu.sync_copy(ax2_h.at[pl.ds(base, _NPW)], ax2)
    pltpu.sync_copy(ay2_h.at[pl.ds(base, _NPW)], ay2)
    pltpu.sync_copy(gt_h, gt)
    pltpu.sync_copy(meta_h, metav)

    mv = metav[...]
    m_h = mv[0]
    m_w = mv[1]

    for r in range(4):
        for k in range(_GP // 16):
            vec = gt[pl.ds(r * _GP + k * 16, 16)]
            for l in range(16):
                j = k * 16 + l
                if j < _G:
                    sgt[r * _G + j] = vec[l]

    def initj(k, _):
        cm[pl.ds(k * 16, 16)] = jnp.full((16,), -2.0, f32)
        ca[pl.ds(k * 16, 16)] = jnp.zeros((16,), i32)
        return 0

    lax.fori_loop(0, _GP, initj, 0)

    def zt(k, _):
        hitb[pl.ds(k * 16, 16)] = jnp.zeros((16,), f32)
        return 0

    lax.fori_loop(0, _NV, zt, 0)
    sidx[...] = jnp.full((16,), _A, i32)

    def vi_body(vp, _):
        oa = vp * 32
        ob = oa + 16
        a1a = ax1[pl.ds(oa, 16)]
        a2a = ay1[pl.ds(oa, 16)]
        a3a = ax2[pl.ds(oa, 16)]
        a4a = ay2[pl.ds(oa, 16)]
        a1b = ax1[pl.ds(ob, 16)]
        a2b = ay1[pl.ds(ob, 16)]
        a3b = ax2[pl.ds(ob, 16)]
        a4b = ay2[pl.ds(ob, 16)]
        areaa = (a3a - a1a + 1.0) * (a4a - a2a + 1.0)
        areab = (a3b - a1b + 1.0) * (a4b - a2b + 1.0)
        inswa = (jnp.where(a1a >= 0.0, 1, 0) * jnp.where(a2a >= 0.0, 1, 0)
                 * jnp.where(a3a < m_w, 1, 0) * jnp.where(a4a < m_h, 1, 0))
        inswb = (jnp.where(a1b >= 0.0, 1, 0) * jnp.where(a2b >= 0.0, 1, 0)
                 * jnp.where(a3b < m_w, 1, 0) * jnp.where(a4b < m_h, 1, 0))
        ins[pl.ds(oa, 16)] = inswa
        ins[pl.ds(ob, 16)] = inswb
        aidxa = base + oa + iota
        aidxb = base + ob + iota

        def j_body(jh, carry):
            bva, bja, bvb, bjb = carry
            for j in (jh * 2, jh * 2 + 1):
                gx1 = sgt[j]
                gy1 = sgt[_G + j]
                gx2 = sgt[2 * _G + j]
                gy2 = sgt[3 * _G + j]
                gw = gx2 - gx1 + 1.0
                gh = gy2 - gy1 + 1.0
                garea = gw * gh
                iwa = jnp.minimum(a3a, gx2) - jnp.maximum(a1a, gx1) + 1.0
                iha = jnp.minimum(a4a, gy2) - jnp.maximum(a2a, gy1) + 1.0
                intera = jnp.maximum(iwa, 0.0) * jnp.maximum(iha, 0.0)
                ioua = intera / (areaa + garea - intera)
                maskeda = jnp.where(inswa > 0, ioua, -1.0)
                iwb = jnp.minimum(a3b, gx2) - jnp.maximum(a1b, gx1) + 1.0
                ihb = jnp.minimum(a4b, gy2) - jnp.maximum(a2b, gy1) + 1.0
                interb = jnp.maximum(iwb, 0.0) * jnp.maximum(ihb, 0.0)
                ioub = interb / (areab + garea - interb)
                maskedb = jnp.where(inswb > 0, ioub, -1.0)
                ca_ = maskeda > bva
                bva = jnp.where(ca_, maskeda, bva)
                bja = jnp.where(ca_, j, bja)
                cb_ = maskedb > bvb
                bvb = jnp.where(cb_, maskedb, bvb)
                bjb = jnp.where(cb_, j, bjb)
                jo = j * 16
                cmo = cm[pl.ds(jo, 16)]
                cao = ca[pl.ds(jo, 16)]
                cca = maskeda > cmo
                cmo = jnp.where(cca, maskeda, cmo)
                cao = jnp.where(cca, aidxa, cao)
                ccb = maskedb > cmo
                cm[pl.ds(jo, 16)] = jnp.where(ccb, maskedb, cmo)
                ca[pl.ds(jo, 16)] = jnp.where(ccb, aidxb, cao)
            return (bva, bja, bvb, bjb)

        z16f = jnp.full((16,), -2.0, f32)
        z16i = jnp.zeros((16,), i32)
        bva, bja, bvb, bjb = lax.fori_loop(
            0, _G // 2, j_body, (z16f, z16i, z16f, z16i))
        bv[pl.ds(oa, 16)] = bva
        bj[pl.ds(oa, 16)] = bja
        bv[pl.ds(ob, 16)] = bvb
        bj[pl.ds(ob, 16)] = bjb
        return 0

    lax.fori_loop(0, _NV // 2, vi_body, 0)

    pltpu.sync_copy(cm, sh_cm.at[pl.ds(wid * _GP * 16, _GP * 16)])
    pltpu.sync_copy(ca, sh_ca.at[pl.ds(wid * _GP * 16, _GP * 16)])
    plsc.subcore_barrier()

    @pl.when(wid < _NMERGE)
    def _merge():
        j0 = wid * 8

        def fetch(s, _):
            pltpu.sync_copy(sh_cm.at[pl.ds(s * _GP * 16 + wid * 128, 128)],
                            mbv.at[pl.ds(s * 128, 128)])
            pltpu.sync_copy(sh_ca.at[pl.ds(s * _GP * 16 + wid * 128, 128)],
                            mba.at[pl.ds(s * 128, 128)])
            return 0

        lax.fori_loop(0, _NW, fetch, 0)
        idxv = jnp.full((16,), _A, i32)
        for jj in range(8):
            def mbody(s, carry):
                gmax, garg = carry
                vv = mbv[pl.ds(s * 128 + jj * 16, 16)]
                va = mba[pl.ds(s * 128 + jj * 16, 16)]
                cc = vv > gmax
                return (jnp.where(cc, vv, gmax), jnp.where(cc, va, garg))

            gmax, garg = lax.fori_loop(
                0, _NW, mbody,
                (jnp.full((16,), -3.0, f32), jnp.zeros((16,), i32)))
            mx = plsc.cummax(gmax)[15]
            am = -plsc.cummax(jnp.where(gmax == mx, -garg, -_A))[15]
            sel = (jnp.where(iota == jj, 1, 0)
                   * jnp.where(j0 + jj < _G, 1, 0))
            idxv = jnp.where(sel > 0, am, idxv)
        sidx[...] = idxv

    pltpu.sync_copy(sidx, sh_win.at[pl.ds(wid * 16, 16)])
    plsc.subcore_barrier()

    pltpu.sync_copy(sh_win, winb)
    ones16 = jnp.ones((16,), f32)
    for r in range(_NMERGE):
        wv = winb[pl.ds(r * 16, 16)]
        loc = wv - base
        mask = (jnp.where(loc >= 0, 1, 0) * jnp.where(loc < _NPW, 1, 0)) > 0
        locc = jnp.clip(loc, 0, _NPW - 1)
        plsc.store_scatter(hitb, [locc], ones16, mask=mask)

    def lab_body(vi, _):
        o = vi * 16
        insw = ins[pl.ds(o, 16)]
        bvv = bv[pl.ds(o, 16)]
        hv = hitb[pl.ds(o, 16)]
        lv = jnp.where(insw * jnp.where(bvv < _NEG_OVERLAP, 1, 0) > 0,
                       0.0, -1.0)
        lv = jnp.where(hv > 0.5, 1.0, lv)
        lv = jnp.where(insw * jnp.where(bvv >= _POS_OVERLAP, 1, 0) > 0,
                       1.0, lv)
        lv = jnp.where(insw > 0, lv, -1.0)
        labb[pl.ds(o, 16)] = lv
        fgb[pl.ds(o, 16)] = jnp.where(lv == 1.0, 1, 0)
        bgb[pl.ds(o, 16)] = jnp.where(lv == 0.0, 1, 0)
        return 0

    lax.fori_loop(0, _NV, lab_body, 0)
    pltpu.sync_copy(fgb, sh_ff.at[pl.ds(base, _NPW)])
    pltpu.sync_copy(bgb, sh_fb.at[pl.ds(base, _NPW)])
    plsc.subcore_barrier()

    @pl.when(wid == 0)
    def _subsample():
        def one_pass(ord_h, sh_flag, target):
            pltpu.sync_copy(ord_h, bigord)
            pltpu.sync_copy(sh_flag, bigflag)

            def kbody(k4, carry):
                k = k4 * 8
                idxs = [bigord[pl.ds((k + u) * 16, 16)] for u in range(8)]
                fs = [plsc.load_gather(bigflag, [ix]) for ix in idxs]
                cums = [plsc.cumsum(f) for f in fs]
                for u in range(8):
                    rank = cums[u] + carry
                    kept = (jnp.where(fs[u] > 0, 1, 0)
                            * jnp.where(rank <= target, 1, 0))
                    plsc.store_scatter(bigflag, [idxs[u]], kept)
                    carry = carry + cums[u][15]
                return carry

            total = lax.fori_loop(0, _KV // 8, kbody, jnp.int32(0))
            pltpu.sync_copy(bigflag, sh_flag)
            return total

        nf = one_pass(ordf_h, sh_ff, jnp.int32(_NUM_FG))
        nbg = _RPN_BATCHSIZE - jnp.minimum(nf, _NUM_FG)
        one_pass(ordb_h, sh_fb, nbg)

    plsc.subcore_barrier()

    pltpu.sync_copy(sh_ff.at[pl.ds(base, _NPW)], kfb)
    pltpu.sync_copy(sh_fb.at[pl.ds(base, _NPW)], kbb)

    def fin_body(vi, _):
        o = vi * 16
        lv = labb[pl.ds(o, 16)]
        kf_ = kfb[pl.ds(o, 16)]
        kb_ = kbb[pl.ds(o, 16)]
        lv = jnp.where(jnp.where(lv == 1.0, 1, 0) * jnp.where(kf_ == 0, 1, 0)
                       > 0, -1.0, lv)
        lv = jnp.where(jnp.where(lv == 0.0, 1, 0) * jnp.where(kb_ == 0, 1, 0)
                       > 0, -1.0, lv)
        labb[pl.ds(o, 16)] = lv
        bjv = bj[pl.ds(o, 16)]
        gx1 = plsc.load_gather(gt, [bjv])
        gy1 = plsc.load_gather(gt, [bjv + _GP])
        gx2 = plsc.load_gather(gt, [bjv + 2 * _GP])
        gy2 = plsc.load_gather(gt, [bjv + 3 * _GP])
        gw = gx2 - gx1 + 1.0
        gh = gy2 - gy1 + 1.0
        gcx = gx1 + 0.5 * gw
        gcy = gy1 + 0.5 * gh
        a1 = ax1[pl.ds(o, 16)]
        a2 = ay1[pl.ds(o, 16)]
        a3 = ax2[pl.ds(o, 16)]
        a4 = ay2[pl.ds(o, 16)]
        aw = a3 - a1 + 1.0
        ah = a4 - a2 + 1.0
        acx = a1 + 0.5 * aw
        acy = a2 + 0.5 * ah
        insw = ins[pl.ds(o, 16)]
        dxb[pl.ds(o, 16)] = jnp.where(insw > 0, (gcx - acx) / aw, 0.0)
        dyb[pl.ds(o, 16)] = jnp.where(insw > 0, (gcy - acy) / ah, 0.0)
        dwb[pl.ds(o, 16)] = jnp.where(insw > 0, _plog(gw / aw), 0.0)
        dhb[pl.ds(o, 16)] = jnp.where(insw > 0, _plog(gh / ah), 0.0)
        return 0

    lax.fori_loop(0, _NV, fin_body, 0)

    pltpu.sync_copy(labb, lab_h.at[pl.ds(base, _NPW)])
    pltpu.sync_copy(dxb, dx_h.at[pl.ds(base, _NPW)])
    pltpu.sync_copy(dyb, dy_h.at[pl.ds(base, _NPW)])
    pltpu.sync_copy(dwb, dw_h.at[pl.ds(base, _NPW)])
    pltpu.sync_copy(dhb, dh_h.at[pl.ds(base, _NPW)])


def _make_sc_kernel():
    f32 = jnp.float32
    i32 = jnp.int32
    return pl.kernel(
        _sc_body,
        out_type=[jax.ShapeDtypeStruct((_A,), f32) for _ in range(5)],
        mesh=plsc.VectorSubcoreMesh(core_axis_name="c", subcore_axis_name="s",
                                    num_cores=1),
        compiler_params=pltpu.CompilerParams(needs_layout_passes=False),
        scratch_types=[
            pltpu.VMEM((_NPW,), f32),
            pltpu.VMEM((_NPW,), f32),
            pltpu.VMEM((_NPW,), f32),
            pltpu.VMEM((_NPW,), f32),
            pltpu.VMEM((_NPW,), i32),
            pltpu.VMEM((_NPW,), f32),
            pltpu.VMEM((_NPW,), i32),
            pltpu.VMEM((_GP * 16,), f32),
            pltpu.VMEM((_GP * 16,), i32),
            pltpu.VMEM((4 * _GP,), f32),
            pltpu.VMEM((16,), f32),
            pltpu.VMEM((_NW * 128,), f32),
            pltpu.VMEM((_NW * 128,), i32),
            pltpu.VMEM((16,), i32),
            pltpu.VMEM((_NW * 16,), i32),
            pltpu.VMEM((_NPW,), f32),
            pltpu.VMEM((_NPW,), f32),
            pltpu.VMEM((_NPW,), i32),
            pltpu.VMEM((_NPW,), i32),
            pltpu.VMEM((_NPW,), i32),
            pltpu.VMEM((_NPW,), i32),
            pltpu.VMEM((_NPW,), f32),
            pltpu.VMEM((_NPW,), f32),
            pltpu.VMEM((_NPW,), f32),
            pltpu.VMEM((_NPW,), f32),
            pltpu.VMEM((_A,), i32),
            pltpu.VMEM((_A,), i32),
            pltpu.SMEM((4 * _G,), f32),
            pltpu.VMEM_SHARED((_NW * _GP * 16,), f32),
            pltpu.VMEM_SHARED((_NW * _GP * 16,), i32),
            pltpu.VMEM_SHARED((_NW * 16,), i32),
            pltpu.VMEM_SHARED((_A,), i32),
            pltpu.VMEM_SHARED((_A,), i32),
        ],
    )


def kernel(scores, gt_boxes, metadata):
    del scores
    f32 = jnp.float32
    gtt = jnp.zeros((4, _GP), f32).at[:, :_G].set(
        gt_boxes.T.astype(f32)).reshape(-1)
    metap = jnp.zeros((16,), f32).at[:3].set(metadata.astype(f32))
    fn = _make_sc_kernel()
    lab, dx, dy, dw, dh = fn(
        jnp.asarray(_ANCHORS[:, 0]), jnp.asarray(_ANCHORS[:, 1]),
        jnp.asarray(_ANCHORS[:, 2]), jnp.asarray(_ANCHORS[:, 3]),
        jnp.asarray(_ORD_F), jnp.asarray(_ORD_B), gtt, metap)
    return jnp.stack([lab, dx, dy, dw, dh], axis=1)


# speedup vs baseline: 1.1755x; 1.0211x over previous
import numpy as np
import jax
import jax.numpy as jnp
from jax import lax
from jax.experimental import pallas as pl
from jax.experimental.pallas import tpu as pltpu
from jax.experimental.pallas import tpu_sc as plsc

_STRIDE = 16
_NEG_OVERLAP = 0.3
_POS_OVERLAP = 0.7
_RPN_BATCHSIZE = 256
_NUM_FG = 128
_FH = _FW = 64
_G = 100
_GP = 112
_A = _FH * _FW * 9
_NW = 16
_NPW = _A // _NW
_NV = _NPW // 16
_KV = _A // 16
_NMERGE = 13


def _np_base_anchors(base_size=16, ratios=(0.5, 1.0, 2.0), scales=(8, 16, 32)):
    base = np.array([1, 1, base_size, base_size], dtype=np.float32) - 1
    w = base[2] - base[0] + 1
    h = base[3] - base[1] + 1
    x_ctr = base[0] + 0.5 * (w - 1)
    y_ctr = base[1] + 0.5 * (h - 1)
    size = w * h
    anchors = []
    for r in ratios:
        size_r = size / r
        ws = np.round(np.sqrt(size_r))
        hs = np.round(ws * r)
        for s in scales:
            ws2 = ws * s
            hs2 = hs * s
            anchors.append([x_ctr - 0.5 * (ws2 - 1), y_ctr - 0.5 * (hs2 - 1),
                            x_ctr + 0.5 * (ws2 - 1), y_ctr + 0.5 * (hs2 - 1)])
    return np.array(anchors, dtype=np.float32)


def _np_all_anchors(fh, fw, stride, base):
    sx = np.arange(fw, dtype=np.float32) * stride
    sy = np.arange(fh, dtype=np.float32) * stride
    sx, sy = np.meshgrid(sx, sy)
    shifts = np.stack([sx.ravel(), sy.ravel(), sx.ravel(), sy.ravel()],
                      axis=1).astype(np.float32)
    all_a = base[None, :, :] + shifts[:, None, :]
    return all_a.reshape(-1, 4)


_ANCHORS = _np_all_anchors(_FH, _FW, _STRIDE, _np_base_anchors())


def _tf2x32(k1, k2, x0, x1):
    def rotl(x, d):
        return (x << np.uint32(d)) | (x >> np.uint32(32 - d))
    ks0 = np.uint32(k1)
    ks1 = np.uint32(k2)
    ks2 = ks0 ^ ks1 ^ np.uint32(0x1BD11BDA)
    x0 = (x0 + ks0).astype(np.uint32)
    x1 = (x1 + ks1).astype(np.uint32)
    rots = ((13, 15, 26, 6), (17, 29, 16, 24))
    inject = ((ks1, ks2, 1), (ks2, ks0, 2), (ks0, ks1, 3),
              (ks1, ks2, 4), (ks2, ks0, 5))
    for blk in range(5):
        for r in rots[blk % 2]:
            x0 = (x0 + x1).astype(np.uint32)
            x1 = rotl(x1, r)
            x1 = x1 ^ x0
        a, b, c = inject[blk]
        x0 = (x0 + a).astype(np.uint32)
        x1 = (x1 + b + np.uint32(c)).astype(np.uint32)
    return x0, x1


def _key42_uniforms(n):
    b1, b2 = _tf2x32(0, 42, np.zeros(2, np.uint32),
                     np.arange(2, dtype=np.uint32))
    out = []
    for (c1, c2) in ((b1[0], b2[0]), (b1[1], b2[1])):
        h1, h2 = _tf2x32(c1, c2, np.zeros(n, np.uint32),
                         np.arange(n, dtype=np.uint32))
        bits = h1 ^ h2
        f = ((bits >> np.uint32(9)) | np.uint32(0x3F800000)).view(np.float32)
        out.append(np.maximum(np.float32(0.0), f - np.float32(1.0)))
    return out


_RND_F, _RND_B = _key42_uniforms(_A)
_ORD_F = np.argsort(_RND_F, kind="stable").astype(np.int32)
_ORD_B = np.argsort(_RND_B, kind="stable").astype(np.int32)

_mgrid = np.linspace(1.0, 2.0, 8193)
_LOG2C = [float(c) for c in np.polyfit(_mgrid, np.log2(_mgrid), 7)]
_LN2 = float(np.log(2.0))


def _plog(x):
    b = lax.bitcast_convert_type(x, jnp.int32)
    e = (b >> 23) - 127
    m = lax.bitcast_convert_type((b & 0x007FFFFF) | 0x3F800000, jnp.float32)
    p = jnp.float32(_LOG2C[0])
    for c in _LOG2C[1:]:
        p = p * m + jnp.float32(c)
    return (e.astype(jnp.float32) + p) * jnp.float32(_LN2)


def _sc_body(ax1_h, ay1_h, ax2_h, ay2_h, ordf_h, ordb_h, gt_h, meta_h,
             lab_h, dx_h, dy_h, dw_h, dh_h,
             ax1, ay1, ax2, ay2, ins, bv, bj, cm, ca, gt, metav,
             mbv, mba, sidx, winb, hitb, labb, fgb, bgb, kfb, kbb,
             dxb, dyb, dwb, dhb, bigord, bigflag, sgt,
             sh_cm, sh_ca, sh_win, sh_ff, sh_fb):
    f32 = jnp.float32
    i32 = jnp.int32
    wid = lax.axis_index("s")
    base = wid * _NPW
    iota = lax.iota(i32, 16)

    pltpu.sync_copy(ax1_h.at[pl.ds(base, _NPW)], ax1)
    pltpu.sync_copy(ay1_h.at[pl.ds(base, _NPW)], ay1)
    pltpu.sync_copy(ax2_h.at[pl.ds(base, _NPW)], ax2)
    pltpu.sync_copy(ay2_h.at[pl.ds(base, _NPW)], ay2)
    pltpu.sync_copy(gt_h, gt)
    pltpu.sync_copy(meta_h, metav)

    mv = metav[...]
    m_h = mv[0]
    m_w = mv[1]

    for r in range(4):
        for k in range(_GP // 16):
            vec = gt[pl.ds(r * _GP + k * 16, 16)]
            for l in range(16):
                j = k * 16 + l
                if j < _G:
                    sgt[r * _G + j] = vec[l]

    def initj(k, _):
        cm[pl.ds(k * 16, 16)] = jnp.full((16,), -2.0, f32)
        ca[pl.ds(k * 16, 16)] = jnp.zeros((16,), i32)
        return 0

    lax.fori_loop(0, _GP, initj, 0)

    def zt(k, _):
        hitb[pl.ds(k * 16, 16)] = jnp.zeros((16,), f32)
        return 0

    lax.fori_loop(0, _NV, zt, 0)
    sidx[...] = jnp.full((16,), _A, i32)

    def vi_body(vp, _):
        oa = vp * 32
        ob = oa + 16
        a1a = ax1[pl.ds(oa, 16)]
        a2a = ay1[pl.ds(oa, 16)]
        a3a = ax2[pl.ds(oa, 16)]
        a4a = ay2[pl.ds(oa, 16)]
        a1b = ax1[pl.ds(ob, 16)]
        a2b = ay1[pl.ds(ob, 16)]
        a3b = ax2[pl.ds(ob, 16)]
        a4b = ay2[pl.ds(ob, 16)]
        areaa = (a3a - a1a + 1.0) * (a4a - a2a + 1.0)
        areab = (a3b - a1b + 1.0) * (a4b - a2b + 1.0)
        inswa = (jnp.where(a1a >= 0.0, 1, 0) * jnp.where(a2a >= 0.0, 1, 0)
                 * jnp.where(a3a < m_w, 1, 0) * jnp.where(a4a < m_h, 1, 0))
        inswb = (jnp.where(a1b >= 0.0, 1, 0) * jnp.where(a2b >= 0.0, 1, 0)
                 * jnp.where(a3b < m_w, 1, 0) * jnp.where(a4b < m_h, 1, 0))
        ins[pl.ds(oa, 16)] = inswa
        ins[pl.ds(ob, 16)] = inswb
        aidxa = base + oa + iota
        aidxb = base + ob + iota

        def j_body(jh, carry):
            bva, bja, bvb, bjb = carry
            for j in (jh * 2, jh * 2 + 1):
                gx1 = sgt[j]
                gy1 = sgt[_G + j]
                gx2 = sgt[2 * _G + j]
                gy2 = sgt[3 * _G + j]
                gw = gx2 - gx1 + 1.0
                gh = gy2 - gy1 + 1.0
                garea = gw * gh
                iwa = jnp.minimum(a3a, gx2) - jnp.maximum(a1a, gx1) + 1.0
                iha = jnp.minimum(a4a, gy2) - jnp.maximum(a2a, gy1) + 1.0
                intera = jnp.maximum(iwa, 0.0) * jnp.maximum(iha, 0.0)
                ioua = intera / (areaa + garea - intera)
                maskeda = jnp.where(inswa > 0, ioua, -1.0)
                iwb = jnp.minimum(a3b, gx2) - jnp.maximum(a1b, gx1) + 1.0
                ihb = jnp.minimum(a4b, gy2) - jnp.maximum(a2b, gy1) + 1.0
                interb = jnp.maximum(iwb, 0.0) * jnp.maximum(ihb, 0.0)
                ioub = interb / (areab + garea - interb)
                maskedb = jnp.where(inswb > 0, ioub, -1.0)
                ca_ = maskeda > bva
                bva = jnp.where(ca_, maskeda, bva)
                bja = jnp.where(ca_, j, bja)
                cb_ = maskedb > bvb
                bvb = jnp.where(cb_, maskedb, bvb)
                bjb = jnp.where(cb_, j, bjb)
                jo = j * 16
                cmo = cm[pl.ds(jo, 16)]
                cao = ca[pl.ds(jo, 16)]
                cca = maskeda > cmo
                cmo = jnp.where(cca, maskeda, cmo)
                cao = jnp.where(cca, aidxa, cao)
                ccb = maskedb > cmo
                cm[pl.ds(jo, 16)] = jnp.where(ccb, maskedb, cmo)
                ca[pl.ds(jo, 16)] = jnp.where(ccb, aidxb, cao)
            return (bva, bja, bvb, bjb)

        z16f = jnp.full((16,), -2.0, f32)
        z16i = jnp.zeros((16,), i32)
        bva, bja, bvb, bjb = lax.fori_loop(
            0, _G // 2, j_body, (z16f, z16i, z16f, z16i))
        bv[pl.ds(oa, 16)] = bva
        bj[pl.ds(oa, 16)] = bja
        bv[pl.ds(ob, 16)] = bvb
        bj[pl.ds(ob, 16)] = bjb
        return 0

    lax.fori_loop(0, _NV // 2, vi_body, 0)

    pltpu.sync_copy(cm, sh_cm.at[pl.ds(wid * _GP * 16, _GP * 16)])
    pltpu.sync_copy(ca, sh_ca.at[pl.ds(wid * _GP * 16, _GP * 16)])
    plsc.subcore_barrier()

    @pl.when(wid < _NMERGE)
    def _merge():
        j0 = wid * 8

        def fetch(s, _):
            pltpu.sync_copy(sh_cm.at[pl.ds(s * _GP * 16 + wid * 128, 128)],
                            mbv.at[pl.ds(s * 128, 128)])
            pltpu.sync_copy(sh_ca.at[pl.ds(s * _GP * 16 + wid * 128, 128)],
                            mba.at[pl.ds(s * 128, 128)])
            return 0

        lax.fori_loop(0, _NW, fetch, 0)
        idxv = jnp.full((16,), _A, i32)
        for jj in range(8):
            def mbody(s, carry):
                gmax, garg = carry
                vv = mbv[pl.ds(s * 128 + jj * 16, 16)]
                va = mba[pl.ds(s * 128 + jj * 16, 16)]
                cc = vv > gmax
                return (jnp.where(cc, vv, gmax), jnp.where(cc, va, garg))

            gmax, garg = lax.fori_loop(
                0, _NW, mbody,
                (jnp.full((16,), -3.0, f32), jnp.zeros((16,), i32)))
            mx = plsc.cummax(gmax)[15]
            am = -plsc.cummax(jnp.where(gmax == mx, -garg, -_A))[15]
            sel = (jnp.where(iota == jj, 1, 0)
                   * jnp.where(j0 + jj < _G, 1, 0))
            idxv = jnp.where(sel > 0, am, idxv)
        sidx[...] = idxv

    pltpu.sync_copy(sidx, sh_win.at[pl.ds(wid * 16, 16)])
    plsc.subcore_barrier()

    pltpu.sync_copy(sh_win, winb)
    ones16 = jnp.ones((16,), f32)
    for r in range(_NMERGE):
        wv = winb[pl.ds(r * 16, 16)]
        loc = wv - base
        mask = (jnp.where(loc >= 0, 1, 0) * jnp.where(loc < _NPW, 1, 0)) > 0
        locc = jnp.clip(loc, 0, _NPW - 1)
        plsc.store_scatter(hitb, [locc], ones16, mask=mask)

    def lab_body(vi, _):
        o = vi * 16
        insw = ins[pl.ds(o, 16)]
        bvv = bv[pl.ds(o, 16)]
        hv = hitb[pl.ds(o, 16)]
        lv = jnp.where(insw * jnp.where(bvv < _NEG_OVERLAP, 1, 0) > 0,
                       0.0, -1.0)
        lv = jnp.where(hv > 0.5, 1.0, lv)
        lv = jnp.where(insw * jnp.where(bvv >= _POS_OVERLAP, 1, 0) > 0,
                       1.0, lv)
        lv = jnp.where(insw > 0, lv, -1.0)
        labb[pl.ds(o, 16)] = lv
        fgb[pl.ds(o, 16)] = jnp.where(lv == 1.0, 1, 0)
        bgb[pl.ds(o, 16)] = jnp.where(lv == 0.0, 1, 0)
        return 0

    lax.fori_loop(0, _NV, lab_body, 0)
    pltpu.sync_copy(fgb, sh_ff.at[pl.ds(base, _NPW)])
    pltpu.sync_copy(bgb, sh_fb.at[pl.ds(base, _NPW)])
    plsc.subcore_barrier()

    def one_pass(ord_h, sh_flag, target):
        pltpu.sync_copy(ord_h, bigord)
        pltpu.sync_copy(sh_flag, bigflag)

        def kbody(k4, carry):
            k = k4 * 8
            idxs = [bigord[pl.ds((k + u) * 16, 16)] for u in range(8)]
            fs = [plsc.load_gather(bigflag, [ix]) for ix in idxs]
            cums = [plsc.cumsum(f) for f in fs]
            for u in range(8):
                rank = cums[u] + carry
                kept = (jnp.where(fs[u] > 0, 1, 0)
                        * jnp.where(rank <= target, 1, 0))
                plsc.store_scatter(bigflag, [idxs[u]], kept)
                carry = carry + cums[u][15]
            return carry

        lax.fori_loop(0, _KV // 8, kbody, jnp.int32(0))
        pltpu.sync_copy(bigflag, sh_flag)

    @pl.when(wid == 0)
    def _subsample_fg():
        one_pass(ordf_h, sh_ff, jnp.int32(_NUM_FG))

    @pl.when(wid == 1)
    def _subsample_bg():
        pltpu.sync_copy(sh_ff, bigflag)

        def cbody(k4, acc):
            acc = acc + bigflag[pl.ds(k4 * 16, 16)]
            return acc

        acc = lax.fori_loop(0, _KV, cbody, jnp.zeros((16,), i32))
        nf = plsc.cumsum(acc)[15]
        nbg = _RPN_BATCHSIZE - jnp.minimum(nf, _NUM_FG)
        one_pass(ordb_h, sh_fb, nbg)

    plsc.subcore_barrier()

    pltpu.sync_copy(sh_ff.at[pl.ds(base, _NPW)], kfb)
    pltpu.sync_copy(sh_fb.at[pl.ds(base, _NPW)], kbb)

    def fin_body(vi, _):
        o = vi * 16
        lv = labb[pl.ds(o, 16)]
        kf_ = kfb[pl.ds(o, 16)]
        kb_ = kbb[pl.ds(o, 16)]
        lv = jnp.where(jnp.where(lv == 1.0, 1, 0) * jnp.where(kf_ == 0, 1, 0)
                       > 0, -1.0, lv)
        lv = jnp.where(jnp.where(lv == 0.0, 1, 0) * jnp.where(kb_ == 0, 1, 0)
                       > 0, -1.0, lv)
        labb[pl.ds(o, 16)] = lv
        bjv = bj[pl.ds(o, 16)]
        gx1 = plsc.load_gather(gt, [bjv])
        gy1 = plsc.load_gather(gt, [bjv + _GP])
        gx2 = plsc.load_gather(gt, [bjv + 2 * _GP])
        gy2 = plsc.load_gather(gt, [bjv + 3 * _GP])
        gw = gx2 - gx1 + 1.0
        gh = gy2 - gy1 + 1.0
        gcx = gx1 + 0.5 * gw
        gcy = gy1 + 0.5 * gh
        a1 = ax1[pl.ds(o, 16)]
        a2 = ay1[pl.ds(o, 16)]
        a3 = ax2[pl.ds(o, 16)]
        a4 = ay2[pl.ds(o, 16)]
        aw = a3 - a1 + 1.0
        ah = a4 - a2 + 1.0
        acx = a1 + 0.5 * aw
        acy = a2 + 0.5 * ah
        insw = ins[pl.ds(o, 16)]
        dxb[pl.ds(o, 16)] = jnp.where(insw > 0, (gcx - acx) / aw, 0.0)
        dyb[pl.ds(o, 16)] = jnp.where(insw > 0, (gcy - acy) / ah, 0.0)
        dwb[pl.ds(o, 16)] = jnp.where(insw > 0, _plog(gw / aw), 0.0)
        dhb[pl.ds(o, 16)] = jnp.where(insw > 0, _plog(gh / ah), 0.0)
        return 0

    lax.fori_loop(0, _NV, fin_body, 0)

    pltpu.sync_copy(labb, lab_h.at[pl.ds(base, _NPW)])
    pltpu.sync_copy(dxb, dx_h.at[pl.ds(base, _NPW)])
    pltpu.sync_copy(dyb, dy_h.at[pl.ds(base, _NPW)])
    pltpu.sync_copy(dwb, dw_h.at[pl.ds(base, _NPW)])
    pltpu.sync_copy(dhb, dh_h.at[pl.ds(base, _NPW)])


def _make_sc_kernel():
    f32 = jnp.float32
    i32 = jnp.int32
    return pl.kernel(
        _sc_body,
        out_type=[jax.ShapeDtypeStruct((_A,), f32) for _ in range(5)],
        mesh=plsc.VectorSubcoreMesh(core_axis_name="c", subcore_axis_name="s",
                                    num_cores=1),
        compiler_params=pltpu.CompilerParams(needs_layout_passes=False),
        scratch_types=[
            pltpu.VMEM((_NPW,), f32),
            pltpu.VMEM((_NPW,), f32),
            pltpu.VMEM((_NPW,), f32),
            pltpu.VMEM((_NPW,), f32),
            pltpu.VMEM((_NPW,), i32),
            pltpu.VMEM((_NPW,), f32),
            pltpu.VMEM((_NPW,), i32),
            pltpu.VMEM((_GP * 16,), f32),
            pltpu.VMEM((_GP * 16,), i32),
            pltpu.VMEM((4 * _GP,), f32),
            pltpu.VMEM((16,), f32),
            pltpu.VMEM((_NW * 128,), f32),
            pltpu.VMEM((_NW * 128,), i32),
            pltpu.VMEM((16,), i32),
            pltpu.VMEM((_NW * 16,), i32),
            pltpu.VMEM((_NPW,), f32),
            pltpu.VMEM((_NPW,), f32),
            pltpu.VMEM((_NPW,), i32),
            pltpu.VMEM((_NPW,), i32),
            pltpu.VMEM((_NPW,), i32),
            pltpu.VMEM((_NPW,), i32),
            pltpu.VMEM((_NPW,), f32),
            pltpu.VMEM((_NPW,), f32),
            pltpu.VMEM((_NPW,), f32),
            pltpu.VMEM((_NPW,), f32),
            pltpu.VMEM((_A,), i32),
            pltpu.VMEM((_A,), i32),
            pltpu.SMEM((4 * _G,), f32),
            pltpu.VMEM_SHARED((_NW * _GP * 16,), f32),
            pltpu.VMEM_SHARED((_NW * _GP * 16,), i32),
            pltpu.VMEM_SHARED((_NW * 16,), i32),
            pltpu.VMEM_SHARED((_A,), i32),
            pltpu.VMEM_SHARED((_A,), i32),
        ],
    )


def kernel(scores, gt_boxes, metadata):
    del scores
    f32 = jnp.float32
    gtt = jnp.zeros((4, _GP), f32).at[:, :_G].set(
        gt_boxes.T.astype(f32)).reshape(-1)
    metap = jnp.zeros((16,), f32).at[:3].set(metadata.astype(f32))
    fn = _make_sc_kernel()
    lab, dx, dy, dw, dh = fn(
        jnp.asarray(_ANCHORS[:, 0]), jnp.asarray(_ANCHORS[:, 1]),
        jnp.asarray(_ANCHORS[:, 2]), jnp.asarray(_ANCHORS[:, 3]),
        jnp.asarray(_ORD_F), jnp.asarray(_ORD_B), gtt, metap)
    return jnp.stack([lab, dx, dy, dw, dh], axis=1)
